# baseline (speedup 1.0000x reference)
import functools
import numpy as np
import jax
from jax import lax
import jax.numpy as jnp
from jax.experimental import pallas as pl
from jax.experimental.pallas import tpu as pltpu
from jax.experimental.pallas import tpu_sc as plsc

_SLOW_FRAMES = 8
_QUARTERS = 4


def _sc_gather(frames):
    C, T, H, W = frames.shape
    S = _SLOW_FRAMES
    NC, NS = 2, 16
    NW = NC * NS
    n_tasks = C * S * _QUARTERS
    assert n_tasks % NW == 0
    per_w = n_tasks // NW
    rpt = H // _QUARTERS

    mesh = plsc.VectorSubcoreMesh(core_axis_name="c", subcore_axis_name="s")

    @functools.partial(
        pl.kernel,
        out_type=jax.ShapeDtypeStruct((C, S, H, W), frames.dtype),
        mesh=mesh,
        scratch_types=[
            pltpu.VMEM((2, rpt, W), frames.dtype),
            pltpu.SemaphoreType.DMA((2,)),
            pltpu.SemaphoreType.DMA((2,)),
        ],
    )
    def run(frames_hbm, slow_hbm, buf, rsem, wsem):
        wid = lax.axis_index("s") * NC + lax.axis_index("c")

        def slices(k):
            task = wid * per_w + k
            c = task // (S * _QUARTERS)
            j = (task // _QUARTERS) % S
            q = task % _QUARTERS
            t = (j * (T - 1)) // (S - 1)
            src = frames_hbm.at[c, t, pl.ds(q * rpt, rpt)]
            dst = slow_hbm.at[c, j, pl.ds(q * rpt, rpt)]
            return src, dst

        src0, dst0 = slices(0)
        src1, dst1 = slices(1)
        src2, dst2 = slices(2)
        r0 = pltpu.make_async_copy(src0, buf.at[0], rsem.at[0])
        r1 = pltpu.make_async_copy(src1, buf.at[1], rsem.at[1])
        r2 = pltpu.make_async_copy(src2, buf.at[0], rsem.at[0])
        w0 = pltpu.make_async_copy(buf.at[0], dst0, wsem.at[0])
        w1 = pltpu.make_async_copy(buf.at[1], dst1, wsem.at[1])
        w2 = pltpu.make_async_copy(buf.at[0], dst2, wsem.at[0])
        r0.start()
        r1.start()
        r0.wait()
        w0.start()
        r1.wait()
        w1.start()
        w0.wait()
        r2.start()
        r2.wait()
        w2.start()
        w1.wait()
        w2.wait()

    return run(frames)


def kernel(frames):
    slow = _sc_gather(frames)
    return (slow, frames)


# speedup vs baseline: 2.2506x; 2.2506x over previous
import numpy as np
import jax
import jax.numpy as jnp
from jax.experimental import pallas as pl
from jax.experimental.pallas import tpu as pltpu

_SLOW_FRAMES = 8
_NBUF = 8
_LOOKAHEAD = 4


def _make_body(idx, T):
    slot_of = {t: j for j, t in enumerate(idx)}

    def _body(frames_ref, slow_ref, fast_ref, buf, rsem, fsem, ssem):
        reads, fwrites, swrites = {}, {}, {}
        for t in range(T):
            b = t % _NBUF
            reads[t] = pltpu.make_async_copy(
                frames_ref.at[:, t:t + 1], buf.at[b], rsem.at[b]
            )
            fwrites[t] = pltpu.make_async_copy(
                buf.at[b], fast_ref.at[:, t:t + 1], fsem.at[b]
            )
            if t in slot_of:
                j = slot_of[t]
                swrites[t] = pltpu.make_async_copy(
                    buf.at[b], slow_ref.at[:, j:j + 1], ssem.at[j % _NBUF]
                )

        for step in range(T + _LOOKAHEAD):
            t = step
            if t < T:
                if t >= _NBUF:
                    fwrites[t - _NBUF].wait()
                    if (t - _NBUF) in swrites:
                        swrites[t - _NBUF].wait()
                reads[t].start()
            u = step - _LOOKAHEAD
            if u >= 0:
                reads[u].wait()
                fwrites[u].start()
                if u in swrites:
                    swrites[u].start()
        for t in range(T - _NBUF, T):
            fwrites[t].wait()
            if t in swrites:
                swrites[t].wait()

    return _body


def kernel(frames):
    C, T, H, W = frames.shape
    idx = [int(v) for v in np.linspace(0.0, float(T - 1), _SLOW_FRAMES).astype(np.int32)]

    slow, fast = pl.pallas_call(
        _make_body(idx, T),
        in_specs=[pl.BlockSpec(memory_space=pltpu.MemorySpace.HBM)],
        out_specs=(
            pl.BlockSpec(memory_space=pltpu.MemorySpace.HBM),
            pl.BlockSpec(memory_space=pltpu.MemorySpace.HBM),
        ),
        out_shape=(
            jax.ShapeDtypeStruct((C, _SLOW_FRAMES, H, W), frames.dtype),
            jax.ShapeDtypeStruct((C, T, H, W), frames.dtype),
        ),
        scratch_shapes=[
            pltpu.VMEM((_NBUF, C, 1, H, W), frames.dtype),
            pltpu.SemaphoreType.DMA((_NBUF,)),
            pltpu.SemaphoreType.DMA((_NBUF,)),
            pltpu.SemaphoreType.DMA((_NBUF,)),
        ],
    )(frames)
    return (slow, fast)
